# baseline (speedup 1.0000x reference)
import functools

import jax
import jax.numpy as jnp
from jax import lax
from jax.experimental import pallas as pl
from jax.experimental.pallas import tpu as pltpu, tpu_sc as plsc

N = 10000
D = 128
G = 64
E = 320000

NC = 2
NS = 16
CHUNK = 64
NCH = 160
EP = NC * NS * NCH * CHUNK
EPT = EP // (NC * NS)
NBUF = 4
DCH = 128
DNCH = EPT // DCH
RPT = 624
RREM = N - NS * RPT

_mesh = plsc.VectorSubcoreMesh(core_axis_name="c", subcore_axis_name="s")


def _deg_body(dst_h, ew_h, zeros_h, out_h, dstv, ewv, zv, acc):
    cid = lax.axis_index("c")
    sid = lax.axis_index("s")
    pltpu.sync_copy(zeros_h.at[pl.ds(0, RPT)], zv)
    pltpu.sync_copy(zv, acc.at[pl.ds(sid * RPT, RPT)])

    @pl.when(sid == NS - 1)
    def _():
        pltpu.sync_copy(zv.at[pl.ds(0, RREM)], acc.at[pl.ds(NS * RPT, RREM)])

    plsc.subcore_barrier()
    tile_base = cid * (EP // 2) + sid * EPT

    def chunk(i, carry):
        base = tile_base + i * DCH
        pltpu.sync_copy(dst_h.at[pl.ds(base, DCH)], dstv)
        pltpu.sync_copy(ew_h.at[pl.ds(base, DCH)], ewv)
        pltpu.sync_copy(ewv, acc.at[dstv], add=True)
        return carry

    lax.fori_loop(0, DNCH, chunk, 0)
    plsc.subcore_barrier()
    pltpu.sync_copy(acc.at[pl.ds(sid * RPT, RPT)], zv)
    pltpu.sync_copy(zv, out_h.at[pl.ds(cid * N + sid * RPT, RPT)])

    @pl.when(sid == NS - 1)
    def _():
        pltpu.sync_copy(acc.at[pl.ds(NS * RPT, RREM)], zv.at[pl.ds(0, RREM)])
        pltpu.sync_copy(zv.at[pl.ds(0, RREM)], out_h.at[pl.ds(cid * N + NS * RPT, RREM)])


_deg_kernel = functools.partial(
    pl.kernel,
    out_type=jax.ShapeDtypeStruct((NC * N,), jnp.float32),
    mesh=_mesh,
    scratch_types=[
        pltpu.VMEM((DCH,), jnp.int32),
        pltpu.VMEM((DCH,), jnp.float32),
        pltpu.VMEM((RPT,), jnp.float32),
        pltpu.VMEM_SHARED((N,), jnp.float32),
    ],
)(_deg_body)


def _make_agg():
    pieces = [(64 * k, 64) for k in range(9)] + [(576, 48)]

    def body(g_h, src_h, dst_h, ew3_h, zeros_h, out_h,
             ew_all, sb0, sb1, sb2, sb3, db0, db1, db2, db3,
             r0, r1, r2, r3, acc,
             p0, p1, p2, p3, g0, g1, g2, g3, s0, s1, s2, s3):
        cid = lax.axis_index("c")
        sid = lax.axis_index("s")
        w = cid * NS + sid
        sbufs = [sb0, sb1, sb2, sb3]
        dbufs = [db0, db1, db2, db3]
        rbufs = [r0, r1, r2, r3]
        psems = [p0, p1, p2, p3]
        gsems = [g0, g1, g2, g3]
        ssems = [s0, s1, s2, s3]

        pltpu.sync_copy(ew3_h.at[w], ew_all)

        pltpu.sync_copy(zeros_h.at[pl.ds(0, CHUNK)], r0)
        for off, size in pieces:
            pltpu.sync_copy(r0.at[pl.ds(0, size)],
                            acc.at[pl.ds(sid * RPT + off, size)])

        @pl.when(sid == NS - 1)
        def _():
            pltpu.sync_copy(r0.at[pl.ds(0, RREM)], acc.at[pl.ds(NS * RPT, RREM)])

        plsc.subcore_barrier()

        def issue_idx(i, b):
            base = w * EPT + i * CHUNK
            pltpu.async_copy(src_h.at[pl.ds(base, CHUNK)], sbufs[b], psems[b])
            pltpu.async_copy(dst_h.at[pl.ds(base, CHUNK)], dbufs[b], psems[b])

        def wait_idx(i, b):
            base = w * EPT + i * CHUNK
            pltpu.make_async_copy(src_h.at[pl.ds(base, CHUNK)], sbufs[b], psems[b]).wait()
            pltpu.make_async_copy(dst_h.at[pl.ds(base, CHUNK)], dbufs[b], psems[b]).wait()

        def start_gather(i, b):
            pltpu.async_copy(g_h.at[sbufs[b]], rbufs[b], gsems[b])

        def wait_gather(i, b):
            pltpu.make_async_copy(g_h.at[sbufs[b]], rbufs[b], gsems[b]).wait()

        def start_scatter(i, b):
            pltpu.async_copy(rbufs[b], acc.at[dbufs[b]], ssems[b], add=True)

        def wait_scatter(i, b):
            pltpu.make_async_copy(rbufs[b], acc.at[dbufs[b]], ssems[b]).wait()

        def scale(i, b):
            rows = rbufs[b]

            ih = i // 2
            il = i % 2

            def group(k16, c2):
                ewvec = ew_all[ih, pl.ds(il * CHUNK + k16 * 16, 16)]
                for t in range(16):
                    s = lax.gather(
                        ewvec, jnp.full((16, 1), t, jnp.int32),
                        lax.GatherDimensionNumbers(
                            offset_dims=(), collapsed_slice_dims=(0,),
                            start_index_map=(0,)),
                        slice_sizes=(1,),
                        mode=lax.GatherScatterMode.PROMISE_IN_BOUNDS)
                    k = k16 * 16 + t
                    for j in range(D // 16):
                        sl = pl.ds(j * 16, 16)
                        rows[k, sl] = rows[k, sl] * s
                return c2

            lax.fori_loop(0, CHUNK // 16, group, 0)

        for b in range(3):
            issue_idx(b, b)
        for b in range(2):
            wait_idx(b, b)
            start_gather(b, b)

        def outer(kk, carry):
            for b in range(NBUF):
                i = kk * NBUF + b
                wait_gather(i, b)
                scale(i, b)
                start_scatter(i, b)
                nxt = i + 3
                bj = (b + 3) % NBUF

                @pl.when(nxt < NCH)
                def _():
                    @pl.when(i >= 1)
                    def _():
                        wait_scatter(i - 1, bj)

                    issue_idx(nxt, bj)

                nx2 = i + 2
                b2 = (b + 2) % NBUF

                @pl.when(nx2 < NCH)
                def _():
                    wait_idx(nx2, b2)
                    start_gather(nx2, b2)

            return carry

        lax.fori_loop(0, NCH // NBUF, outer, 0)
        for b in range(NBUF):
            i = NCH - NBUF + b
            wait_scatter(i, i % NBUF)
        plsc.subcore_barrier()
        for off, size in pieces:
            pltpu.sync_copy(acc.at[pl.ds(sid * RPT + off, size)],
                            r0.at[pl.ds(0, size)])
            pltpu.sync_copy(r0.at[pl.ds(0, size)],
                            out_h.at[cid, pl.ds(sid * RPT + off, size)])

        @pl.when(sid == NS - 1)
        def _():
            pltpu.sync_copy(acc.at[pl.ds(NS * RPT, RREM)], r0.at[pl.ds(0, RREM)])
            pltpu.sync_copy(r0.at[pl.ds(0, RREM)], out_h.at[cid, pl.ds(NS * RPT, RREM)])

    return pl.kernel(
        body,
        out_type=jax.ShapeDtypeStruct((NC, N, D), jnp.float32),
        mesh=_mesh,
        scratch_types=(
            [pltpu.VMEM((NCH // 2, 2 * CHUNK), jnp.float32)]
            + [pltpu.VMEM((CHUNK,), jnp.int32) for _ in range(8)]
            + [pltpu.VMEM((CHUNK, D), jnp.float32) for _ in range(4)]
            + [pltpu.VMEM_SHARED((N, D), jnp.float32)]
            + [pltpu.SemaphoreType.DMA for _ in range(12)]
        ),
    )


_agg_kernel = _make_agg()


def _g1_body(x_ref, w_ref, d0_ref, d1_ref, o_ref):
    deg = d0_ref[...] + d1_ref[...] + 1.0
    dinv = lax.rsqrt(deg)
    o_ref[...] = jnp.dot(x_ref[...], w_ref[...], preferred_element_type=jnp.float32) * dinv


def _layer_body(a0_ref, a1_ref, g_ref, d0_ref, d1_ref, b_ref, w_ref, o_ref):
    deg = d0_ref[...] + d1_ref[...] + 1.0
    dinv = lax.rsqrt(deg)
    h = jnp.maximum(dinv * (a0_ref[...] + a1_ref[...] + g_ref[...]) + b_ref[...], 0.0)
    o_ref[...] = jnp.dot(h, w_ref[...], preferred_element_type=jnp.float32) * dinv


def _final_body(a0_ref, a1_ref, g_ref, d0_ref, d1_ref, b_ref, wh_ref, bh_ref,
                batch_ref, o_ref):
    deg = d0_ref[...] + d1_ref[...] + 1.0
    dinv = lax.rsqrt(deg)
    h = jnp.maximum(dinv * (a0_ref[...] + a1_ref[...] + g_ref[...]) + b_ref[...], 0.0)
    iota = lax.broadcasted_iota(jnp.int32, (G, N), 0)
    mask = (batch_ref[...] == iota).astype(jnp.float32)
    sums = jnp.dot(mask, h, preferred_element_type=jnp.float32)
    cnts = jnp.sum(mask, axis=1, keepdims=True)
    pooled = sums / jnp.maximum(cnts, 1.0)
    o_ref[...] = jnp.dot(pooled, wh_ref[...], preferred_element_type=jnp.float32) + bh_ref[...]


def _tc_g1(x, w, d0, d1):
    return pl.pallas_call(
        _g1_body, out_shape=jax.ShapeDtypeStruct((N, D), jnp.float32),
    )(x, w, d0, d1)


def _tc_layer(a0, a1, g, d0, d1, b, w):
    return pl.pallas_call(
        _layer_body, out_shape=jax.ShapeDtypeStruct((N, D), jnp.float32),
    )(a0, a1, g, d0, d1, b, w)


def _tc_final(a0, a1, g, d0, d1, b, wh, bh, batch):
    return pl.pallas_call(
        _final_body, out_shape=jax.ShapeDtypeStruct((G, 1), jnp.float32),
    )(a0, a1, g, d0, d1, b, wh, bh, batch)


def kernel(x, edge_index, edge_weight, batch, W1, b1, W2, b2, Wh, bh):
    src = edge_index[0]
    dst = edge_index[1]
    pad = EP - E
    zi = jnp.zeros((pad,), jnp.int32)
    src_p = jnp.concatenate([src, zi])
    dst_p = jnp.concatenate([dst, zi])
    ew_p = jnp.concatenate([edge_weight, jnp.zeros((pad,), jnp.float32)])
    zeros_rows = jnp.zeros((N, D), jnp.float32)
    zeros_n = jnp.zeros((N,), jnp.float32)

    ew3 = ew_p.reshape(NC * NS, NCH // 2, 2 * CHUNK)

    dparts = _deg_kernel(dst_p, ew_p, zeros_n)
    d0 = dparts[:N].reshape(N, 1)
    d1 = dparts[N:].reshape(N, 1)

    g1 = _tc_g1(x, W1, d0, d1)
    a1 = _agg_kernel(g1, src_p, dst_p, ew3, zeros_rows)
    g2 = _tc_layer(a1[0], a1[1], g1, d0, d1, b1.reshape(1, D), W2)
    a2 = _agg_kernel(g2, src_p, dst_p, ew3, zeros_rows)
    return _tc_final(a2[0], a2[1], g2, d0, d1, b2.reshape(1, D), Wh,
                     bh.reshape(1, 1), batch.reshape(1, N))
